# baseline (speedup 1.0000x reference)
import jax
import jax.numpy as jnp
from jax.experimental import pallas as pl

NUM_VISIBLE = 1024
NUM_EXPERTS = 16
HIDDEN_DIM = 256
ENERGY_SCALE = 3.0

BT = 512


def _fused_kernel(x_ref, wenc_ref, benc_ref, wh_ref, t_ref, beta_ref,
                  p_ref, e_ref, l_ref):
    enc = jnp.tanh(
        jnp.dot(x_ref[...].astype(jnp.bfloat16), wenc_ref[...],
                preferred_element_type=jnp.float32)
        + benc_ref[...])
    s = jnp.dot(enc, wh_ref[...], preferred_element_type=jnp.float32)
    en = ENERGY_SCALE * jnp.tanh(s + t_ref[...])
    lg = (-beta_ref[0, 0]) * en
    m = jnp.max(lg, axis=-1, keepdims=True)
    ex = jnp.exp(lg - m)
    p_ref[...] = ex / jnp.sum(ex, axis=-1, keepdims=True)
    e_ref[...] = en
    l_ref[...] = lg


def kernel(x, W_enc, b_enc, W_e, b_e, inv_temp):
    B = x.shape[0]
    H = HIDDEN_DIM
    E = NUM_EXPERTS
    W_enc = W_enc.astype(jnp.bfloat16)
    w_h = W_e[:H]
    t = (W_e[H:, 0] + b_e).reshape(1, E)
    beta = jax.nn.softplus(inv_temp).reshape(1, 1)
    b_enc2 = b_enc.reshape(1, H)

    grid = (B // BT,)
    out_shape = [jax.ShapeDtypeStruct((B, E), jnp.float32)] * 3
    probs, energies, logits = pl.pallas_call(
        _fused_kernel,
        grid=grid,
        in_specs=[
            pl.BlockSpec((BT, NUM_VISIBLE), lambda i: (i, 0)),
            pl.BlockSpec((NUM_VISIBLE, H), lambda i: (0, 0)),
            pl.BlockSpec((1, H), lambda i: (0, 0)),
            pl.BlockSpec((H, 1), lambda i: (0, 0)),
            pl.BlockSpec((1, E), lambda i: (0, 0)),
            pl.BlockSpec((1, 1), lambda i: (0, 0)),
        ],
        out_specs=[pl.BlockSpec((BT, E), lambda i: (i, 0))] * 3,
        out_shape=out_shape,
    )(x, W_enc, b_enc2, w_h, t, beta)
    return (probs, energies, logits)


# speedup vs baseline: 1.0441x; 1.0441x over previous
import jax
import jax.numpy as jnp
from jax.experimental import pallas as pl
from jax.experimental.pallas import tpu as pltpu

NUM_VISIBLE = 1024
NUM_EXPERTS = 16
HIDDEN_DIM = 256
ENERGY_SCALE = 3.0

BT = 512


def _fused_kernel(x_ref, wenc_ref, benc_ref, wh_ref, t_ref, beta_ref,
                  p_ref, e_ref, l_ref):
    enc = jnp.tanh(
        jnp.dot(x_ref[...], wenc_ref[...], preferred_element_type=jnp.float32)
        + benc_ref[...])
    s = jnp.dot(enc, wh_ref[...], preferred_element_type=jnp.float32)
    en = ENERGY_SCALE * jnp.tanh(s + t_ref[...])
    lg = (-beta_ref[0, 0]) * en
    m = jnp.max(lg, axis=-1, keepdims=True)
    ex = jnp.exp(lg - m)
    p_ref[...] = ex / jnp.sum(ex, axis=-1, keepdims=True)
    e_ref[...] = en
    l_ref[...] = lg


def kernel(x, W_enc, b_enc, W_e, b_e, inv_temp):
    B = x.shape[0]
    H = HIDDEN_DIM
    E = NUM_EXPERTS
    w_h = W_e[:H]
    t = (W_e[H:, 0] + b_e).reshape(1, E)
    beta = jax.nn.softplus(inv_temp).reshape(1, 1)
    b_enc2 = b_enc.reshape(1, H)

    grid = (B // BT,)
    out_shape = [jax.ShapeDtypeStruct((B, E), jnp.float32)] * 3
    probs, energies, logits = pl.pallas_call(
        _fused_kernel,
        grid=grid,
        in_specs=[
            pl.BlockSpec((BT, NUM_VISIBLE), lambda i: (i, 0)),
            pl.BlockSpec((NUM_VISIBLE, H), lambda i: (0, 0)),
            pl.BlockSpec((1, H), lambda i: (0, 0)),
            pl.BlockSpec((H, 1), lambda i: (0, 0)),
            pl.BlockSpec((1, E), lambda i: (0, 0)),
            pl.BlockSpec((1, 1), lambda i: (0, 0)),
        ],
        out_specs=[pl.BlockSpec((BT, E), lambda i: (i, 0))] * 3,
        out_shape=out_shape,
        compiler_params=pltpu.CompilerParams(
            dimension_semantics=("parallel",)),
    )(x, W_enc, b_enc2, w_h, t, beta)
    return (probs, energies, logits)


# speedup vs baseline: 1.2583x; 1.2052x over previous
import jax
import jax.numpy as jnp
from jax.experimental import pallas as pl
from jax.experimental.pallas import tpu as pltpu

NUM_VISIBLE = 1024
NUM_EXPERTS = 16
HIDDEN_DIM = 256
ENERGY_SCALE = 3.0

BT = 512


def _fused_kernel(x_ref, wenc_ref, benc_ref, we_ref, be_ref, it_ref,
                  p_ref, e_ref, l_ref):
    H = HIDDEN_DIM
    we = we_ref[...]
    wh = we[:, :H]
    wi = we[:, H:]
    enc = jnp.tanh(
        jnp.dot(x_ref[...], wenc_ref[...], preferred_element_type=jnp.float32)
        + benc_ref[...])
    s = jnp.sum(enc * wh, axis=1, keepdims=True)
    beta = jax.nn.softplus(it_ref[0, 0])
    en = ENERGY_SCALE * jnp.tanh(s + (wi + be_ref[0, 0]))
    lg = (-beta) * en
    m = jnp.max(lg, axis=-1, keepdims=True)
    ex = jnp.exp(lg - m)
    p_ref[...] = ex / jnp.sum(ex, axis=-1, keepdims=True)
    e_ref[...] = en
    l_ref[...] = lg


def kernel(x, W_enc, b_enc, W_e, b_e, inv_temp):
    B = x.shape[0]
    H = HIDDEN_DIM
    E = NUM_EXPERTS
    we_row = W_e.reshape(1, H + E)
    b_enc2 = b_enc.reshape(1, H)
    be2 = b_e.reshape(1, 1)
    it2 = inv_temp.reshape(1, 1)

    grid = (B // BT,)
    out_shape = [jax.ShapeDtypeStruct((B, E), jnp.float32)] * 3
    probs, energies, logits = pl.pallas_call(
        _fused_kernel,
        grid=grid,
        in_specs=[
            pl.BlockSpec((BT, NUM_VISIBLE), lambda i: (i, 0)),
            pl.BlockSpec((NUM_VISIBLE, H), lambda i: (0, 0)),
            pl.BlockSpec((1, H), lambda i: (0, 0)),
            pl.BlockSpec((1, H + E), lambda i: (0, 0)),
            pl.BlockSpec((1, 1), lambda i: (0, 0)),
            pl.BlockSpec((1, 1), lambda i: (0, 0)),
        ],
        out_specs=[pl.BlockSpec((BT, E), lambda i: (i, 0))] * 3,
        out_shape=out_shape,
        compiler_params=pltpu.CompilerParams(
            dimension_semantics=("parallel",)),
    )(x, W_enc, b_enc2, we_row, be2, it2)
    return (probs, energies, logits)


# speedup vs baseline: 1.4703x; 1.1685x over previous
import jax
import jax.numpy as jnp
from jax.experimental import pallas as pl
from jax.experimental.pallas import tpu as pltpu

NUM_VISIBLE = 1024
NUM_EXPERTS = 16
HIDDEN_DIM = 256
ENERGY_SCALE = 3.0

BT = 1024


def _fused_kernel(x_ref, wenc_ref, benc_ref, we_ref, be_ref, it_ref,
                  p_ref, e_ref, l_ref):
    H = HIDDEN_DIM
    we = we_ref[...]
    wh = we[:, :H]
    wi = we[:, H:]
    enc = jnp.tanh(
        jnp.dot(x_ref[...], wenc_ref[...], preferred_element_type=jnp.float32)
        + benc_ref[...])
    s = jnp.sum(enc * wh, axis=1, keepdims=True)
    beta = jax.nn.softplus(it_ref[0, 0])
    en = ENERGY_SCALE * jnp.tanh(s + (wi + be_ref[0, 0]))
    lg = (-beta) * en
    m = jnp.max(lg, axis=-1, keepdims=True)
    ex = jnp.exp(lg - m)
    p_ref[...] = ex / jnp.sum(ex, axis=-1, keepdims=True)
    e_ref[...] = en
    l_ref[...] = lg


def kernel(x, W_enc, b_enc, W_e, b_e, inv_temp):
    B = x.shape[0]
    H = HIDDEN_DIM
    E = NUM_EXPERTS
    we_row = W_e.reshape(1, H + E)
    b_enc2 = b_enc.reshape(1, H)
    be2 = b_e.reshape(1, 1)
    it2 = inv_temp.reshape(1, 1)

    grid = (B // BT,)
    out_shape = [jax.ShapeDtypeStruct((B, E), jnp.float32)] * 3
    probs, energies, logits = pl.pallas_call(
        _fused_kernel,
        grid=grid,
        in_specs=[
            pl.BlockSpec((BT, NUM_VISIBLE), lambda i: (i, 0)),
            pl.BlockSpec((NUM_VISIBLE, H), lambda i: (0, 0)),
            pl.BlockSpec((1, H), lambda i: (0, 0)),
            pl.BlockSpec((1, H + E), lambda i: (0, 0)),
            pl.BlockSpec((1, 1), lambda i: (0, 0)),
            pl.BlockSpec((1, 1), lambda i: (0, 0)),
        ],
        out_specs=[pl.BlockSpec((BT, E), lambda i: (i, 0))] * 3,
        out_shape=out_shape,
        compiler_params=pltpu.CompilerParams(
            dimension_semantics=("parallel",)),
    )(x, W_enc, b_enc2, we_row, be2, it2)
    return (probs, energies, logits)


# speedup vs baseline: 1.5607x; 1.0615x over previous
import jax
import jax.numpy as jnp
from jax.experimental import pallas as pl
from jax.experimental.pallas import tpu as pltpu

NUM_VISIBLE = 1024
NUM_EXPERTS = 16
HIDDEN_DIM = 256
ENERGY_SCALE = 3.0

BT = 2048


def _fused_kernel(x_ref, wenc_ref, benc_ref, we_ref, be_ref, it_ref,
                  p_ref, e_ref, l_ref):
    H = HIDDEN_DIM
    we = we_ref[...]
    wh = we[:, :H]
    wi = we[:, H:]
    enc = jnp.tanh(
        jnp.dot(x_ref[...], wenc_ref[...], preferred_element_type=jnp.float32)
        + benc_ref[...])
    s = jnp.sum(enc * wh, axis=1, keepdims=True)
    beta = jax.nn.softplus(it_ref[0, 0])
    en = ENERGY_SCALE * jnp.tanh(s + (wi + be_ref[0, 0]))
    lg = (-beta) * en
    m = jnp.max(lg, axis=-1, keepdims=True)
    ex = jnp.exp(lg - m)
    p_ref[...] = ex / jnp.sum(ex, axis=-1, keepdims=True)
    e_ref[...] = en
    l_ref[...] = lg


def kernel(x, W_enc, b_enc, W_e, b_e, inv_temp):
    B = x.shape[0]
    H = HIDDEN_DIM
    E = NUM_EXPERTS
    we_row = W_e.reshape(1, H + E)
    b_enc2 = b_enc.reshape(1, H)
    be2 = b_e.reshape(1, 1)
    it2 = inv_temp.reshape(1, 1)

    grid = (B // BT,)
    out_shape = [jax.ShapeDtypeStruct((B, E), jnp.float32)] * 3
    probs, energies, logits = pl.pallas_call(
        _fused_kernel,
        grid=grid,
        in_specs=[
            pl.BlockSpec((BT, NUM_VISIBLE), lambda i: (i, 0)),
            pl.BlockSpec((NUM_VISIBLE, H), lambda i: (0, 0)),
            pl.BlockSpec((1, H), lambda i: (0, 0)),
            pl.BlockSpec((1, H + E), lambda i: (0, 0)),
            pl.BlockSpec((1, 1), lambda i: (0, 0)),
            pl.BlockSpec((1, 1), lambda i: (0, 0)),
        ],
        out_specs=[pl.BlockSpec((BT, E), lambda i: (i, 0))] * 3,
        out_shape=out_shape,
        compiler_params=pltpu.CompilerParams(
            dimension_semantics=("parallel",)),
    )(x, W_enc, b_enc2, we_row, be2, it2)
    return (probs, energies, logits)
